# baseline (speedup 1.0000x reference)
import functools

import jax
import jax.numpy as jnp
from jax import lax
from jax.experimental import pallas as pl
from jax.experimental.pallas import tpu as pltpu
from jax.experimental.pallas import tpu_sc as plsc

EMB_DIM = 64
LANES = 16
NUM_WORKERS = 32
BLK_B = 128
VGRP = BLK_B // LANES


def _gather_body(seq_len, b_per_w, table_hbm, idx_hbm, out_hbm,
                 idx_all, ridx0, ridx1, woff0, woff1, pair0, pair1,
                 tr0, tr1, sg0, sg1, so0, so1):
    cid = lax.axis_index("c")
    sid = lax.axis_index("s")
    wid = sid * 2 + cid

    ridx = (ridx0, ridx1)
    woff = (woff0, woff1)
    pair = (pair0, pair1)
    tr = (tr0, tr1)
    sem_g = (sg0, sg1)
    sem_o = (so0, so1)

    blk_per_w = b_per_w // BLK_B
    n_blocks = seq_len * blk_per_w

    lane = lax.iota(jnp.int32, LANES)

    pltpu.sync_copy(idx_hbm.at[pl.ds(wid * b_per_w * seq_len,
                                     b_per_w * seq_len)], idx_all)

    def prep_and_fire(k, j):
        s = k // blk_per_w
        bbl = k % blk_per_w
        for v in range(VGRP):
            bloc = bbl * BLK_B + v * LANES
            vals = plsc.load_gather(idx_all, [(bloc + lane) * seq_len + s])
            ridx[j][pl.ds(v * LANES, LANES)] = vals >> 1
            woff[j][pl.ds(v * LANES, LANES)] = (vals & 1) * EMB_DIM
        pltpu.async_copy(table_hbm.at[ridx[j]], pair[j], sem_g[j])

    def wait_gather(j):
        pltpu.make_async_copy(table_hbm.at[ridx[j]], pair[j],
                              sem_g[j]).wait()

    def transpose_select(j):
        def step(d, carry):
            for v in range(VGRP):
                col = woff[j][pl.ds(v * LANES, LANES)] + d
                vals = plsc.load_gather(pair[j], [lane + v * LANES, col])
                tr[j][d, pl.ds(v * LANES, LANES)] = vals
            return carry

        lax.fori_loop(0, EMB_DIM, step, 0, unroll=False)

    def fire_out(k, j):
        s = k // blk_per_w
        bbg = wid * blk_per_w + k % blk_per_w
        for dh in range(EMB_DIM // 8):
            pltpu.async_copy(tr[j].at[pl.ds(dh * 8, 8), :],
                             out_hbm.at[s, pl.ds(dh * 8, 8),
                                        pl.ds(bbg * BLK_B, BLK_B)],
                             sem_o[j])

    def drain_out(j):
        for dh in range(EMB_DIM // 8):
            pltpu.make_async_copy(tr[j].at[pl.ds(dh * 8, 8), :],
                                  out_hbm.at[0, pl.ds(dh * 8, 8),
                                             pl.ds(wid * BLK_B, BLK_B)],
                                  sem_o[j]).wait()

    prep_and_fire(0, 0)

    def body(i, carry):
        for j in (0, 1):
            k = 2 * i + j

            @pl.when(k < n_blocks - 1)
            def _():
                prep_and_fire(k + 1, 1 - j)

            wait_gather(j)

            @pl.when(k > 1)
            def _():
                drain_out(j)

            transpose_select(j)
            fire_out(k, j)
        return carry

    lax.fori_loop(0, n_blocks // 2, body, 0, unroll=False)
    drain_out(0)
    drain_out(1)


def kernel(embeddings, indices):
    batch, seq_len = indices.shape
    flat_idx = indices.reshape(-1).astype(jnp.int32)
    b_per_w = batch // NUM_WORKERS
    assert b_per_w * NUM_WORKERS == batch and b_per_w % BLK_B == 0
    assert (seq_len * b_per_w // BLK_B) % 2 == 0

    table2 = embeddings.reshape(embeddings.shape[0] // 2, 2 * EMB_DIM)

    mesh = plsc.VectorSubcoreMesh(core_axis_name="c", subcore_axis_name="s")
    grab = pl.kernel(
        functools.partial(_gather_body, seq_len, b_per_w),
        out_type=jax.ShapeDtypeStruct((seq_len, EMB_DIM, batch), jnp.float32),
        mesh=mesh,
        scratch_types=[
            pltpu.VMEM((b_per_w * seq_len,), jnp.int32),
            pltpu.VMEM((BLK_B,), jnp.int32),
            pltpu.VMEM((BLK_B,), jnp.int32),
            pltpu.VMEM((BLK_B,), jnp.int32),
            pltpu.VMEM((BLK_B,), jnp.int32),
            pltpu.VMEM((BLK_B, 2 * EMB_DIM), jnp.float32),
            pltpu.VMEM((BLK_B, 2 * EMB_DIM), jnp.float32),
            pltpu.VMEM((EMB_DIM, BLK_B), jnp.float32),
            pltpu.VMEM((EMB_DIM, BLK_B), jnp.float32),
            pltpu.SemaphoreType.DMA,
            pltpu.SemaphoreType.DMA,
            pltpu.SemaphoreType.DMA,
            pltpu.SemaphoreType.DMA,
        ],
        compiler_params=pltpu.CompilerParams(use_tc_tiling_on_sc=True,
                                             needs_layout_passes=False),
    )
    out_t = grab(table2, flat_idx)
    return out_t.transpose(2, 0, 1)


# speedup vs baseline: 1.3822x; 1.3822x over previous
import functools

import jax
import jax.numpy as jnp
from jax import lax
from jax.experimental import pallas as pl
from jax.experimental.pallas import tpu as pltpu
from jax.experimental.pallas import tpu_sc as plsc

EMB_DIM = 64
LANES = 16
NUM_WORKERS = 32
BLK_B = 128
VGRP = BLK_B // LANES


def _gather_body(seq_len, b_per_w, table_hbm, idx_hbm, out_hbm,
                 idx_all, ridx0, ridx1, woff0, woff1, pair0, pair1,
                 tr0, tr1, sg0, sg1, so0, so1):
    cid = lax.axis_index("c")
    sid = lax.axis_index("s")
    wid = sid * 2 + cid

    ridx = (ridx0, ridx1)
    woff = (woff0, woff1)
    pair = (pair0, pair1)
    tr = (tr0, tr1)
    sem_g = (sg0, sg1)
    sem_o = (so0, so1)

    blk_per_w = b_per_w // BLK_B
    n_blocks = seq_len * blk_per_w

    lane = lax.iota(jnp.int32, LANES)

    pltpu.sync_copy(idx_hbm.at[pl.ds(wid * b_per_w * seq_len,
                                     b_per_w * seq_len)], idx_all)

    def prep_and_fire(k, j):
        s = k // blk_per_w
        bbl = k % blk_per_w
        for v in range(VGRP):
            bloc = bbl * BLK_B + v * LANES
            vals = plsc.load_gather(idx_all, [(bloc + lane) * seq_len + s])
            ridx[j][pl.ds(v * LANES, LANES)] = vals >> 1
            woff[j][pl.ds(v * LANES, LANES)] = (vals & 1) * EMB_DIM
        pltpu.async_copy(table_hbm.at[ridx[j]], pair[j], sem_g[j])

    def wait_gather(j):
        pltpu.make_async_copy(table_hbm.at[ridx[j]], pair[j],
                              sem_g[j]).wait()

    def transpose_select(j):
        cols0 = tuple(woff[j][pl.ds(v * LANES, LANES)] for v in range(VGRP))

        def step(d, cols):
            for v in range(VGRP):
                vals = plsc.load_gather(pair[j], [lane + v * LANES, cols[v]])
                tr[j][d, pl.ds(v * LANES, LANES)] = vals
            return tuple(c + 1 for c in cols)

        lax.fori_loop(0, EMB_DIM, step, cols0, unroll=4)

    def fire_out(k, j):
        s = k // blk_per_w
        bbg = wid * blk_per_w + k % blk_per_w
        pltpu.async_copy(tr[j],
                         out_hbm.at[s, :, pl.ds(bbg * BLK_B, BLK_B)],
                         sem_o[j])

    def drain_out(j):
        pltpu.make_async_copy(tr[j],
                              out_hbm.at[0, :, pl.ds(wid * BLK_B, BLK_B)],
                              sem_o[j]).wait()

    prep_and_fire(0, 0)

    def body(i, carry):
        for j in (0, 1):
            k = 2 * i + j

            @pl.when(k < n_blocks - 1)
            def _():
                prep_and_fire(k + 1, 1 - j)

            wait_gather(j)

            @pl.when(k > 1)
            def _():
                drain_out(j)

            transpose_select(j)
            fire_out(k, j)
        return carry

    lax.fori_loop(0, n_blocks // 2, body, 0, unroll=False)
    drain_out(0)
    drain_out(1)


def kernel(embeddings, indices):
    batch, seq_len = indices.shape
    flat_idx = indices.reshape(-1).astype(jnp.int32)
    b_per_w = batch // NUM_WORKERS
    assert b_per_w * NUM_WORKERS == batch and b_per_w % BLK_B == 0
    assert (seq_len * b_per_w // BLK_B) % 2 == 0

    table2 = embeddings.reshape(embeddings.shape[0] // 2, 2 * EMB_DIM)

    mesh = plsc.VectorSubcoreMesh(core_axis_name="c", subcore_axis_name="s")
    grab = pl.kernel(
        functools.partial(_gather_body, seq_len, b_per_w),
        out_type=jax.ShapeDtypeStruct((seq_len, EMB_DIM, batch), jnp.float32),
        mesh=mesh,
        scratch_types=[
            pltpu.VMEM((b_per_w * seq_len,), jnp.int32),
            pltpu.VMEM((BLK_B,), jnp.int32),
            pltpu.VMEM((BLK_B,), jnp.int32),
            pltpu.VMEM((BLK_B,), jnp.int32),
            pltpu.VMEM((BLK_B,), jnp.int32),
            pltpu.VMEM((BLK_B, 2 * EMB_DIM), jnp.float32),
            pltpu.VMEM((BLK_B, 2 * EMB_DIM), jnp.float32),
            pltpu.VMEM((EMB_DIM, BLK_B), jnp.float32),
            pltpu.VMEM((EMB_DIM, BLK_B), jnp.float32),
            pltpu.SemaphoreType.DMA,
            pltpu.SemaphoreType.DMA,
            pltpu.SemaphoreType.DMA,
            pltpu.SemaphoreType.DMA,
        ],
        compiler_params=pltpu.CompilerParams(use_tc_tiling_on_sc=True,
                                             needs_layout_passes=False),
    )
    out_t = grab(table2, flat_idx)
    return out_t.transpose(2, 0, 1)


# speedup vs baseline: 2.0212x; 1.4623x over previous
import functools

import jax
import jax.numpy as jnp
from jax import lax
from jax.experimental import pallas as pl
from jax.experimental.pallas import tpu as pltpu
from jax.experimental.pallas import tpu_sc as plsc

EMB_DIM = 64
NUM_WORKERS = 32
CHUNK = 800


def _gather_body(n_chunks, rows_per_worker, table_hbm, idx_hbm, out_hbm,
                 idx_all, rows0, rows1, sg0, sg1, so0, so1):
    cid = lax.axis_index("c")
    sid = lax.axis_index("s")
    wid = sid * 2 + cid
    base = wid * rows_per_worker

    rows = (rows0, rows1)
    sem_g = (sg0, sg1)
    sem_o = (so0, so1)

    pltpu.sync_copy(idx_hbm.at[pl.ds(base, rows_per_worker)], idx_all)

    def idx_slice(g):
        return idx_all.at[pl.ds(g * CHUNK, CHUNK)]

    def fire_gather(g, j):
        pltpu.async_copy(table_hbm.at[idx_slice(g)], rows[j], sem_g[j])

    def wait_gather(j):
        pltpu.make_async_copy(table_hbm.at[idx_slice(0)], rows[j],
                              sem_g[j]).wait()

    def fire_out(g, j):
        pltpu.async_copy(rows[j], out_hbm.at[pl.ds(base + g * CHUNK, CHUNK)],
                         sem_o[j])

    def wait_out(j):
        pltpu.make_async_copy(rows[j], out_hbm.at[pl.ds(base, CHUNK)],
                              sem_o[j]).wait()

    fire_gather(0, 0)

    def pair(i, carry):
        for j in (0, 1):
            g = 2 * i + j

            @pl.when(g > 0)
            def _():
                wait_out(1 - j)

            @pl.when(g < n_chunks - 1)
            def _():
                fire_gather(g + 1, 1 - j)

            wait_gather(j)
            fire_out(g, j)
        return carry

    lax.fori_loop(0, n_chunks // 2, pair, 0, unroll=False)
    wait_out((n_chunks - 1) % 2)


def kernel(embeddings, indices):
    flat_idx = indices.reshape(-1).astype(jnp.int32)
    total = flat_idx.shape[0]
    rows_per_worker = total // NUM_WORKERS
    n_chunks = rows_per_worker // CHUNK
    assert rows_per_worker * NUM_WORKERS == total
    assert n_chunks * CHUNK == rows_per_worker and n_chunks % 2 == 0

    mesh = plsc.VectorSubcoreMesh(core_axis_name="c", subcore_axis_name="s")
    grab = pl.kernel(
        functools.partial(_gather_body, n_chunks, rows_per_worker),
        out_type=jax.ShapeDtypeStruct((total, EMB_DIM), jnp.float32),
        mesh=mesh,
        scratch_types=[
            pltpu.VMEM((rows_per_worker,), jnp.int32),
            pltpu.VMEM((CHUNK, EMB_DIM), jnp.float32),
            pltpu.VMEM((CHUNK, EMB_DIM), jnp.float32),
            pltpu.SemaphoreType.DMA,
            pltpu.SemaphoreType.DMA,
            pltpu.SemaphoreType.DMA,
            pltpu.SemaphoreType.DMA,
        ],
        compiler_params=pltpu.CompilerParams(use_tc_tiling_on_sc=False),
    )
    out = grab(embeddings, flat_idx)
    return out.reshape(indices.shape + (EMB_DIM,))
